# Initial kernel scaffold; baseline (speedup 1.0000x reference)
import jax
import jax.numpy as jnp
from jax.experimental import pallas as pl


def kernel(A, S, W1, b1, W2, b2, W3, b3):
    raise NotImplementedError("write your pallas kernel here")


# speedup vs baseline: 1.2510x; 1.2510x over previous
import jax
import jax.numpy as jnp
from jax.experimental import pallas as pl


def _gcn3_kernel(a_ref, s_ref, w1_ref, b1_ref, w2_ref, b2_ref, w3_ref,
                 b3_ref, out_ref):
    a = a_ref[0]
    x = s_ref[0]
    outs = []
    for w_ref, b_ref in ((w1_ref, b1_ref), (w2_ref, b2_ref),
                         (w3_ref, b3_ref)):
        t = jnp.dot(a, x, preferred_element_type=jnp.float32)
        x = jnp.maximum(
            jnp.dot(t, w_ref[...], preferred_element_type=jnp.float32)
            + b_ref[...], 0.0)
        outs.append(x)
    out_ref[0] = jnp.concatenate(outs, axis=-1)


def kernel(A, S, W1, b1, W2, b2, W3, b3):
    B, N, _ = A.shape
    D_IN = S.shape[-1]
    D_H = W1.shape[1]
    b1r = b1.reshape(1, D_H)
    b2r = b2.reshape(1, D_H)
    b3r = b3.reshape(1, D_H)

    w_spec = lambda shp: pl.BlockSpec(shp, lambda b: (0,) * len(shp))
    out = pl.pallas_call(
        _gcn3_kernel,
        grid=(B,),
        in_specs=[
            pl.BlockSpec((1, N, N), lambda b: (b, 0, 0)),
            pl.BlockSpec((1, N, D_IN), lambda b: (b, 0, 0)),
            w_spec(W1.shape),
            w_spec(b1r.shape),
            w_spec(W2.shape),
            w_spec(b2r.shape),
            w_spec(W3.shape),
            w_spec(b3r.shape),
        ],
        out_specs=pl.BlockSpec((1, N, 3 * D_H), lambda b: (b, 0, 0)),
        out_shape=jax.ShapeDtypeStruct((B, N, 3 * D_H), jnp.float32),
    )(A, S, W1, b1r, W2, b2r, W3, b3r)
    return out
